# baseline (speedup 1.0000x reference)
import jax
import jax.numpy as jnp
from jax.experimental import pallas as pl
from jax.experimental.pallas import tpu as pltpu

B, N = 256, 512
EMB, VIS, KPT = 128, 1, 17
TOKEN_DIM = 128
M = B * N
SMALL = VIS + 4 + KPT * 3
MASK_COL = SMALL

ROWS = 8192
CHUNKS = 2
BC = B // CHUNKS
MC = BC * N
BLKS_C = MC // ROWS


def _compute(emb_ref, xs_ref, w_emb_ref, w_xs_ref, out_ref):
    xs = xs_ref[...]
    acc = jnp.dot(emb_ref[...], w_emb_ref[...],
                  preferred_element_type=jnp.float32)
    acc = acc + jnp.dot(xs, w_xs_ref[...],
                        preferred_element_type=jnp.float32)
    m = jax.lax.slice(xs, (0, MASK_COL), (ROWS, MASK_COL + 1))
    out_ref[...] = acc * m.astype(jnp.float32)


def _body(emb_ref, xs_ref, w_emb_ref, w_xs_ref, out_ref):
    _compute(emb_ref, xs_ref, w_emb_ref, w_xs_ref, out_ref)


def _body_alias(emb_ref, xs_ref, w_emb_ref, w_xs_ref, prev_ref, out_ref):
    del prev_ref
    _compute(emb_ref, xs_ref, w_emb_ref, w_xs_ref, out_ref)


def _pack(vis, bbox, kpt, mask):
    xs = jnp.concatenate(
        [vis, bbox, kpt.reshape(BC, N, KPT * 3),
         mask[..., None].astype(jnp.float32)],
        axis=-1,
    ).astype(jnp.bfloat16)
    xs = jnp.pad(xs, ((0, 0), (0, 0), (0, 128 - (SMALL + 1))))
    return xs.reshape(MC, 128)


def kernel(feats_masks, embeddings, visibility_scores, bbox_ltwh,
           keypoints_xyc, app_W, app_b, st_W, st_b):
    emb = embeddings.reshape(M, EMB)

    w_emb = app_W[:, :EMB].T
    w_small = jnp.concatenate([app_W[:, EMB:], st_W], axis=1).T
    bias_row = (app_b + st_b).reshape(1, TOKEN_DIM)
    w_xs = jnp.concatenate(
        [w_small, bias_row, jnp.zeros((128 - SMALL - 1, TOKEN_DIM), jnp.float32)],
        axis=0,
    ).astype(jnp.bfloat16)

    xs_chunks = [
        _pack(visibility_scores[c * BC:(c + 1) * BC],
              bbox_ltwh[c * BC:(c + 1) * BC],
              keypoints_xyc[c * BC:(c + 1) * BC],
              feats_masks[c * BC:(c + 1) * BC])
        for c in range(CHUNKS)
    ]

    out = None
    for c in range(CHUNKS):
        off = c * BLKS_C
        in_specs = [
            pl.BlockSpec((ROWS, EMB), lambda i, off=off: (i + off, 0)),
            pl.BlockSpec((ROWS, 128), lambda i: (i, 0)),
            pl.BlockSpec((EMB, TOKEN_DIM), lambda i: (0, 0)),
            pl.BlockSpec((128, TOKEN_DIM), lambda i: (0, 0)),
        ]
        args = [emb, xs_chunks[c], w_emb, w_xs]
        kwargs = {}
        body = _body
        if out is not None:
            in_specs.append(pl.BlockSpec(memory_space=pltpu.MemorySpace.HBM))
            args.append(out)
            kwargs["input_output_aliases"] = {4: 0}
            body = _body_alias
        out = pl.pallas_call(
            body,
            grid=(BLKS_C,),
            in_specs=in_specs,
            out_specs=pl.BlockSpec((ROWS, TOKEN_DIM),
                                   lambda i, off=off: (i + off, 0)),
            out_shape=jax.ShapeDtypeStruct((M, TOKEN_DIM), jnp.float32),
            **kwargs,
        )(*args)
    return out.reshape(B, N, TOKEN_DIM)


# speedup vs baseline: 1.1364x; 1.1364x over previous
import jax
import jax.numpy as jnp
from jax.experimental import pallas as pl

B, N = 256, 512
EMB, VIS, KPT = 128, 1, 17
TOKEN_DIM = 128
M = B * N
SMALL = VIS + 4 + KPT * 3
MASK_COL = SMALL

ROWS = 16384


def _body(emb_ref, xs_ref, w_emb_ref, w_xs_ref, out_ref):
    xs = xs_ref[...]
    acc = jnp.dot(emb_ref[...], w_emb_ref[...],
                  preferred_element_type=jnp.float32)
    acc = acc + jnp.dot(xs, w_xs_ref[...],
                        preferred_element_type=jnp.float32)
    m = jax.lax.slice(xs, (0, MASK_COL), (ROWS, MASK_COL + 1))
    out_ref[...] = acc * m.astype(jnp.float32)


def kernel(feats_masks, embeddings, visibility_scores, bbox_ltwh,
           keypoints_xyc, app_W, app_b, st_W, st_b):
    emb = embeddings.reshape(M, EMB)

    xs = jnp.concatenate(
        [
            visibility_scores,
            bbox_ltwh,
            keypoints_xyc.reshape(B, N, KPT * 3),
            feats_masks[..., None].astype(jnp.float32),
        ],
        axis=-1,
    ).astype(jnp.bfloat16)
    xs = jnp.pad(xs, ((0, 0), (0, 0), (0, 128 - (SMALL + 1))))
    xs = xs.reshape(M, 128)

    w_emb = app_W[:, :EMB].T
    w_small = jnp.concatenate([app_W[:, EMB:], st_W], axis=1).T
    bias_row = (app_b + st_b).reshape(1, TOKEN_DIM)
    w_xs = jnp.concatenate(
        [w_small, bias_row, jnp.zeros((128 - SMALL - 1, TOKEN_DIM), jnp.float32)],
        axis=0,
    ).astype(jnp.bfloat16)

    grid = (M // ROWS,)
    out = pl.pallas_call(
        _body,
        grid=grid,
        in_specs=[
            pl.BlockSpec((ROWS, EMB), lambda i: (i, 0)),
            pl.BlockSpec((ROWS, 128), lambda i: (i, 0)),
            pl.BlockSpec((EMB, TOKEN_DIM), lambda i: (0, 0)),
            pl.BlockSpec((128, TOKEN_DIM), lambda i: (0, 0)),
        ],
        out_specs=pl.BlockSpec((ROWS, TOKEN_DIM), lambda i: (i, 0)),
        out_shape=jax.ShapeDtypeStruct((M, TOKEN_DIM), jnp.float32),
    )(emb, xs, w_emb, w_xs)
    return out.reshape(B, N, TOKEN_DIM)
